# baseline (speedup 1.0000x reference)
import functools

import jax
import jax.numpy as jnp
from jax.experimental import pallas as pl
from jax.experimental.pallas import tpu as pltpu


def _silu(x):
    return x * jax.nn.sigmoid(x)


def _ngraph_kernel(
    inp_ref, n0_ref, ea_ref, eb_ref,
    mw1_ref, mb1_ref, mw2_ref, mb2_ref, mw3_ref, mb3_ref,
    uw1_ref, ub1_ref, uw2_ref, ub2_ref, uw3_ref, ub3_ref,
    iw1_ref, ib1_ref, iw2_ref, ib2_ref,
    ow1_ref, ob1_ref, ow2_ref, ob2_ref,
    out_ref,
    *, n_total, n_in, n_out, ch_n, ch_inp,
):
    n0 = n0_ref[:]
    t = n0[n_total - n_out:, :]
    n_tail = n_total - n_in

    w1a = mw1_ref[:ch_n, :]
    w1b = mw1_ref[ch_n:2 * ch_n, :]
    w1e = mw1_ref[2 * ch_n:, :]
    mb1 = mb1_ref[:]
    mw2 = mw2_ref[:]
    mb2 = mb2_ref[:]

    e1a = jnp.reshape(ea_ref[:], (n_out * n_total, -1)) @ w1e
    e1a = jnp.reshape(e1a, (n_out, n_total, 64))
    e1b = jnp.reshape(eb_ref[:], (n_total * n_out, -1)) @ w1e
    e1b = jnp.reshape(e1b, (n_total, n_out, 64))
    ta = t @ w1a
    tb = t @ w1b

    nb_tail = n0[n_in:, :] @ w1b
    na_tail = n0[n_in:, :] @ w1a

    h1 = _silu(ta[:, None, :] + nb_tail[None, :, :] + e1a[:, n_in:, :] + mb1)
    h2 = _silu(jnp.reshape(h1, (n_out * n_tail, 64)) @ mw2 + mb2)
    sa_tail = jnp.sum(jnp.reshape(h2, (n_out, n_tail, 32)), axis=1)

    h1 = _silu(na_tail[:, None, :] + tb[None, :, :] + e1b[n_in:, :, :] + mb1)
    h2 = _silu(jnp.reshape(h1, (n_tail * n_out, 64)) @ mw2 + mb2)
    sb_tail = jnp.sum(jnp.reshape(h2, (n_tail, n_out, 32)), axis=0)

    nbatch = inp_ref.shape[0]
    for b in range(nbatch):
        hi = _silu(inp_ref[b] @ iw1_ref[:ch_inp, :]
                   + n0[:n_in, :] @ iw1_ref[ch_inp:, :] + ib1_ref[:])
        yi = hi @ iw2_ref[:] + ib2_ref[:]

        nb_head = yi @ w1b
        na_head = yi @ w1a

        h1 = _silu(ta[:, None, :] + nb_head[None, :, :] + e1a[:, :n_in, :] + mb1)
        h2 = _silu(jnp.reshape(h1, (n_out * n_in, 64)) @ mw2 + mb2)
        sa = sa_tail + jnp.sum(jnp.reshape(h2, (n_out, n_in, 32)), axis=1)
        agg_a = sa @ mw3_ref[:, :ch_n] + float(n_total) * mb3_ref[:, :ch_n]

        h1 = _silu(na_head[:, None, :] + tb[None, :, :] + e1b[:n_in, :, :] + mb1)
        h2 = _silu(jnp.reshape(h1, (n_in * n_out, 64)) @ mw2 + mb2)
        sb = sb_tail + jnp.sum(jnp.reshape(h2, (n_in, n_out, 32)), axis=0)
        agg_b = sb @ mw3_ref[:, ch_n:2 * ch_n] + float(n_total) * mb3_ref[:, ch_n:2 * ch_n]

        u = _silu(t @ uw1_ref[:ch_n, :] + agg_a @ uw1_ref[ch_n:2 * ch_n, :]
                  + agg_b @ uw1_ref[2 * ch_n:, :] + ub1_ref[:])
        u = _silu(u @ uw2_ref[:] + ub2_ref[:])
        upd = u @ uw3_ref[:] + ub3_ref[:]
        new_t = jnp.clip(t + upd, -100.0, 100.0)

        ho = _silu(new_t @ ow1_ref[:] + ob1_ref[:])
        out_ref[b] = ho @ ow2_ref[:] + ob2_ref[:]


def kernel(inp, init_nodes, init_edges,
           msg_w1, msg_b1, msg_w2, msg_b2, msg_w3, msg_b3,
           upd_w1, upd_b1, upd_w2, upd_b2, upd_w3, upd_b3,
           ii_w1, ii_b1, ii_w2, ii_b2,
           oi_w1, oi_b1, oi_w2, oi_b2):
    bsz, n_in, ch_inp = inp.shape
    n_total, ch_n = init_nodes.shape
    n_out = 16
    ch_out = oi_w2.shape[1]
    f32 = jnp.float32

    args = [
        inp, init_nodes, init_edges, init_edges,
        msg_w1, jnp.reshape(msg_b1, (1, 1, -1)), msg_w2, msg_b2[None, :],
        msg_w3, msg_b3[None, :],
        upd_w1, upd_b1[None, :], upd_w2, upd_b2[None, :], upd_w3, upd_b3[None, :],
        ii_w1, ii_b1[None, :], ii_w2, ii_b2[None, :],
        oi_w1, oi_b1[None, :], oi_w2, oi_b2[None, :],
    ]
    row_blk = n_total // n_out - 1
    in_specs = [pl.BlockSpec(a.shape, lambda i, nd=a.ndim: (0,) * nd) for a in args]
    in_specs[2] = pl.BlockSpec((n_out, n_total, init_edges.shape[2]),
                               lambda i: (row_blk, 0, 0))
    in_specs[3] = pl.BlockSpec((n_total, n_out, init_edges.shape[2]),
                               lambda i: (0, row_blk, 0))

    body = functools.partial(_ngraph_kernel, n_total=n_total, n_in=n_in,
                             n_out=n_out, ch_n=ch_n, ch_inp=ch_inp)
    return pl.pallas_call(
        body,
        grid=(1,),
        in_specs=in_specs,
        out_specs=pl.BlockSpec((bsz, n_out, ch_out), lambda i: (0, 0, 0)),
        out_shape=jax.ShapeDtypeStruct((bsz, n_out, ch_out), f32),
    )(*args)


# speedup vs baseline: 1.0076x; 1.0076x over previous
import functools

import jax
import jax.numpy as jnp
from jax.experimental import pallas as pl
from jax.experimental.pallas import tpu as pltpu


def _silu(x):
    return 0.5 * x * (jnp.tanh(0.5 * x) + 1.0)


def _ngraph_kernel(
    inp_ref, n0_ref, ea_ref, eb_ref,
    mw1_ref, mb1_ref, mw2_ref, mb2_ref, mw3_ref, mb3_ref,
    uw1_ref, ub1_ref, uw2_ref, ub2_ref, uw3_ref, ub3_ref,
    iw1_ref, ib1_ref, iw2_ref, ib2_ref,
    ow1_ref, ob1_ref, ow2_ref, ob2_ref,
    out_ref,
    *, n_total, n_in, n_out, ch_n, ch_inp,
):
    n0 = n0_ref[:]
    t = n0[n_total - n_out:, :]
    n_tail = n_total - n_in

    w1a = mw1_ref[:ch_n, :]
    w1b = mw1_ref[ch_n:2 * ch_n, :]
    w1e = mw1_ref[2 * ch_n:, :]
    mb1 = mb1_ref[:]
    mw2 = mw2_ref[:]
    mb2 = mb2_ref[:]

    e1a = jnp.reshape(ea_ref[:], (n_out * n_total, -1)) @ w1e
    e1a = jnp.reshape(e1a, (n_out, n_total, 64))
    e1b = jnp.reshape(eb_ref[:], (n_total * n_out, -1)) @ w1e
    e1b = jnp.reshape(e1b, (n_total, n_out, 64))
    ta = t @ w1a
    tb = t @ w1b

    nb_tail = n0[n_in:, :] @ w1b
    na_tail = n0[n_in:, :] @ w1a

    h1 = _silu(ta[:, None, :] + nb_tail[None, :, :] + e1a[:, n_in:, :] + mb1)
    h2 = _silu(jnp.reshape(h1, (n_out * n_tail, 64)) @ mw2 + mb2)
    sa_tail = jnp.sum(jnp.reshape(h2, (n_out, n_tail, 32)), axis=1)

    h1 = _silu(na_tail[:, None, :] + tb[None, :, :] + e1b[n_in:, :, :] + mb1)
    h2 = _silu(jnp.reshape(h1, (n_tail * n_out, 64)) @ mw2 + mb2)
    sb_tail = jnp.sum(jnp.reshape(h2, (n_tail, n_out, 32)), axis=0)

    nbatch = inp_ref.shape[0]
    for b in range(nbatch):
        hi = _silu(inp_ref[b] @ iw1_ref[:ch_inp, :]
                   + n0[:n_in, :] @ iw1_ref[ch_inp:, :] + ib1_ref[:])
        yi = hi @ iw2_ref[:] + ib2_ref[:]

        nb_head = yi @ w1b
        na_head = yi @ w1a

        h1 = _silu(ta[:, None, :] + nb_head[None, :, :] + e1a[:, :n_in, :] + mb1)
        h2 = _silu(jnp.reshape(h1, (n_out * n_in, 64)) @ mw2 + mb2)
        sa = sa_tail + jnp.sum(jnp.reshape(h2, (n_out, n_in, 32)), axis=1)
        agg_a = sa @ mw3_ref[:, :ch_n] + float(n_total) * mb3_ref[:, :ch_n]

        h1 = _silu(na_head[:, None, :] + tb[None, :, :] + e1b[:n_in, :, :] + mb1)
        h2 = _silu(jnp.reshape(h1, (n_in * n_out, 64)) @ mw2 + mb2)
        sb = sb_tail + jnp.sum(jnp.reshape(h2, (n_in, n_out, 32)), axis=0)
        agg_b = sb @ mw3_ref[:, ch_n:2 * ch_n] + float(n_total) * mb3_ref[:, ch_n:2 * ch_n]

        u = _silu(t @ uw1_ref[:ch_n, :] + agg_a @ uw1_ref[ch_n:2 * ch_n, :]
                  + agg_b @ uw1_ref[2 * ch_n:, :] + ub1_ref[:])
        u = _silu(u @ uw2_ref[:] + ub2_ref[:])
        upd = u @ uw3_ref[:] + ub3_ref[:]
        new_t = jnp.clip(t + upd, -100.0, 100.0)

        ho = _silu(new_t @ ow1_ref[:] + ob1_ref[:])
        out_ref[b] = ho @ ow2_ref[:] + ob2_ref[:]


def kernel(inp, init_nodes, init_edges,
           msg_w1, msg_b1, msg_w2, msg_b2, msg_w3, msg_b3,
           upd_w1, upd_b1, upd_w2, upd_b2, upd_w3, upd_b3,
           ii_w1, ii_b1, ii_w2, ii_b2,
           oi_w1, oi_b1, oi_w2, oi_b2):
    bsz, n_in, ch_inp = inp.shape
    n_total, ch_n = init_nodes.shape
    n_out = 16
    ch_out = oi_w2.shape[1]
    f32 = jnp.float32

    args = [
        inp, init_nodes, init_edges, init_edges,
        msg_w1, jnp.reshape(msg_b1, (1, 1, -1)), msg_w2, msg_b2[None, :],
        msg_w3, msg_b3[None, :],
        upd_w1, upd_b1[None, :], upd_w2, upd_b2[None, :], upd_w3, upd_b3[None, :],
        ii_w1, ii_b1[None, :], ii_w2, ii_b2[None, :],
        oi_w1, oi_b1[None, :], oi_w2, oi_b2[None, :],
    ]
    row_blk = n_total // n_out - 1
    in_specs = [pl.BlockSpec(a.shape, lambda i, nd=a.ndim: (0,) * nd) for a in args]
    in_specs[2] = pl.BlockSpec((n_out, n_total, init_edges.shape[2]),
                               lambda i: (row_blk, 0, 0))
    in_specs[3] = pl.BlockSpec((n_total, n_out, init_edges.shape[2]),
                               lambda i: (0, row_blk, 0))

    body = functools.partial(_ngraph_kernel, n_total=n_total, n_in=n_in,
                             n_out=n_out, ch_n=ch_n, ch_inp=ch_inp)
    return pl.pallas_call(
        body,
        grid=(1,),
        in_specs=in_specs,
        out_specs=pl.BlockSpec((bsz, n_out, ch_out), lambda i: (0, 0, 0)),
        out_shape=jax.ShapeDtypeStruct((bsz, n_out, ch_out), f32),
    )(*args)


# speedup vs baseline: 3.3502x; 3.3250x over previous
import functools

import jax
import jax.numpy as jnp
from jax.experimental import pallas as pl
from jax.experimental.pallas import tpu as pltpu


def _silu(x):
    return 0.5 * x * (jnp.tanh(0.5 * x) + 1.0)


def _ngraph_kernel(
    inp_ref, n0_ref, ea_ref, eb_ref,
    mw1_ref, mb1_ref, mw2_ref, mb2_ref, mw3_ref, mb3_ref,
    uw1_ref, ub1_ref, uw2_ref, ub2_ref, uw3_ref, ub3_ref,
    iw1_ref, ib1_ref, iw2_ref, ib2_ref,
    ow1_ref, ob1_ref, ow2_ref, ob2_ref,
    out_ref,
    *, n_total, n_in, n_out, ch_n, ch_inp,
):
    n0 = n0_ref[:]
    t = n0[n_total - n_out:, :]
    n_tail = n_total - n_in

    w1a = mw1_ref[:ch_n, :]
    w1b = mw1_ref[ch_n:2 * ch_n, :]
    w1e = mw1_ref[2 * ch_n:, :]
    mb1 = mb1_ref[:]
    mw2 = mw2_ref[:]
    mb2 = mb2_ref[:]

    e1a = jnp.reshape(ea_ref[:], (n_out * n_total, -1)) @ w1e
    e1a = jnp.reshape(e1a, (n_out, n_total, 64))
    e1b = jnp.reshape(eb_ref[:], (n_total * n_out, -1)) @ w1e
    e1b = jnp.reshape(e1b, (n_total, n_out, 64))
    ta = t @ w1a
    tb = t @ w1b

    nb_tail = n0[n_in:, :] @ w1b
    na_tail = n0[n_in:, :] @ w1a

    h1 = _silu(ta[:, None, :] + nb_tail[None, :, :] + e1a[:, n_in:, :] + mb1)
    h2 = _silu(jnp.reshape(h1, (n_out * n_tail, 64)) @ mw2 + mb2)
    sa_tail = jnp.sum(jnp.reshape(h2, (n_out, n_tail, 32)), axis=1)

    h1 = _silu(na_tail[:, None, :] + tb[None, :, :] + e1b[n_in:, :, :] + mb1)
    h2 = _silu(jnp.reshape(h1, (n_tail * n_out, 64)) @ mw2 + mb2)
    sb_tail = jnp.sum(jnp.reshape(h2, (n_tail, n_out, 32)), axis=0)

    nbatch = inp_ref.shape[0]
    for b in range(nbatch):
        hi = _silu(inp_ref[b] @ iw1_ref[:ch_inp, :]
                   + n0[:n_in, :] @ iw1_ref[ch_inp:, :] + ib1_ref[:])
        yi = hi @ iw2_ref[:] + ib2_ref[:]

        nb_head = yi @ w1b
        na_head = yi @ w1a

        h1 = _silu(ta[:, None, :] + nb_head[None, :, :] + e1a[:, :n_in, :] + mb1)
        h2 = _silu(jnp.reshape(h1, (n_out * n_in, 64)) @ mw2 + mb2)
        sa = sa_tail + jnp.sum(jnp.reshape(h2, (n_out, n_in, 32)), axis=1)
        agg_a = sa @ mw3_ref[:, :ch_n] + float(n_total) * mb3_ref[:, :ch_n]

        h1 = _silu(na_head[:, None, :] + tb[None, :, :] + e1b[:n_in, :, :] + mb1)
        h2 = _silu(jnp.reshape(h1, (n_in * n_out, 64)) @ mw2 + mb2)
        sb = sb_tail + jnp.sum(jnp.reshape(h2, (n_in, n_out, 32)), axis=0)
        agg_b = sb @ mw3_ref[:, ch_n:2 * ch_n] + float(n_total) * mb3_ref[:, ch_n:2 * ch_n]

        u = _silu(t @ uw1_ref[:ch_n, :] + agg_a @ uw1_ref[ch_n:2 * ch_n, :]
                  + agg_b @ uw1_ref[2 * ch_n:, :] + ub1_ref[:])
        u = _silu(u @ uw2_ref[:] + ub2_ref[:])
        upd = u @ uw3_ref[:] + ub3_ref[:]
        new_t = jnp.clip(t + upd, -100.0, 100.0)

        ho = _silu(new_t @ ow1_ref[:] + ob1_ref[:])
        out_ref[b] = ho @ ow2_ref[:] + ob2_ref[:]


def kernel(inp, init_nodes, init_edges,
           msg_w1, msg_b1, msg_w2, msg_b2, msg_w3, msg_b3,
           upd_w1, upd_b1, upd_w2, upd_b2, upd_w3, upd_b3,
           ii_w1, ii_b1, ii_w2, ii_b2,
           oi_w1, oi_b1, oi_w2, oi_b2):
    bsz, n_in, ch_inp = inp.shape
    n_total, ch_n = init_nodes.shape
    n_out = 16
    ch_out = oi_w2.shape[1]
    f32 = jnp.float32

    ea = jax.lax.slice_in_dim(init_edges, n_total - n_out, n_total, axis=0)
    eb = jax.lax.slice_in_dim(init_edges, n_total - n_out, n_total, axis=1)
    args = [
        inp, init_nodes, ea, eb,
        msg_w1, jnp.reshape(msg_b1, (1, 1, -1)), msg_w2, msg_b2[None, :],
        msg_w3, msg_b3[None, :],
        upd_w1, upd_b1[None, :], upd_w2, upd_b2[None, :], upd_w3, upd_b3[None, :],
        ii_w1, ii_b1[None, :], ii_w2, ii_b2[None, :],
        oi_w1, oi_b1[None, :], oi_w2, oi_b2[None, :],
    ]
    in_specs = [pl.BlockSpec(a.shape, lambda i, nd=a.ndim: (0,) * nd) for a in args]

    body = functools.partial(_ngraph_kernel, n_total=n_total, n_in=n_in,
                             n_out=n_out, ch_n=ch_n, ch_inp=ch_inp)
    return pl.pallas_call(
        body,
        grid=(1,),
        in_specs=in_specs,
        out_specs=pl.BlockSpec((bsz, n_out, ch_out), lambda i: (0, 0, 0)),
        out_shape=jax.ShapeDtypeStruct((bsz, n_out, ch_out), f32),
    )(*args)
